# baseline (speedup 1.0000x reference)
import functools

import jax
import jax.numpy as jnp
from jax import lax
from jax.experimental import pallas as pl
from jax.experimental.pallas import tpu as pltpu
from jax.experimental.pallas import tpu_sc as plsc

ROWS = 8192
COLS = 2048
REP = 4
NC = 2
NS = 16
NW = NC * NS
ROWS_PER_W = ROWS // NW
CIN = 4
COUT = CIN * REP
NROUND = ROWS_PER_W // CIN
NPAIR = NROUND // 2


def _sc_kernel(x_hbm, out_hbm, b0, b1, gi0, gi1, gs0, gs1, ss0, ss1):
    wid = lax.axis_index("s") * NC + lax.axis_index("c")
    base0 = wid * ROWS_PER_W
    lane = lax.iota(jnp.int32, 16)
    slots = ((b0, gi0, gs0, ss0), (b1, gi1, gs1, ss1))

    def fire_gather(g, buf, gidx, gsem):
        gidx[...] = base0 + g * CIN + lax.shift_right_logical(lane, 2)
        pltpu.async_copy(x_hbm.at[gidx], buf, gsem)

    for s in range(2):
        fire_gather(s, slots[s][0], slots[s][1], slots[s][2])

    def pair_body(p, carry):
        for s in range(2):
            g = 2 * p + s
            buf, gidx, gsem, ssem = slots[s]
            pltpu.make_async_copy(x_hbm.at[gidx], buf, gsem).wait()
            dst = (base0 + g * CIN) * REP
            pltpu.async_copy(buf, out_hbm.at[pl.ds(dst, COUT)], ssem)
        for s in range(2):
            g = 2 * p + s
            buf, gidx, gsem, ssem = slots[s]
            dst = (base0 + g * CIN) * REP
            pltpu.make_async_copy(
                buf, out_hbm.at[pl.ds(dst, COUT)], ssem).wait()

            @pl.when(g + 2 < NROUND)
            def _():
                fire_gather(g + 2, buf, gidx, gsem)

        return carry

    lax.fori_loop(0, NPAIR, pair_body, 0)


@jax.jit
def _repeat_interleave(x):
    mesh = plsc.VectorSubcoreMesh(core_axis_name="c", subcore_axis_name="s")
    k = functools.partial(
        pl.kernel,
        out_type=jax.ShapeDtypeStruct((ROWS * REP, COLS), jnp.float32),
        mesh=mesh,
        scratch_types=[
            pltpu.VMEM((COUT, COLS), jnp.float32),
            pltpu.VMEM((COUT, COLS), jnp.float32),
            pltpu.VMEM((16,), jnp.int32),
            pltpu.VMEM((16,), jnp.int32),
            pltpu.SemaphoreType.DMA,
            pltpu.SemaphoreType.DMA,
            pltpu.SemaphoreType.DMA,
            pltpu.SemaphoreType.DMA,
        ],
    )(_sc_kernel)
    return k(x)


def kernel(x):
    return _repeat_interleave(x)


# speedup vs baseline: 1.7357x; 1.7357x over previous
import functools

import jax
import jax.numpy as jnp
from jax import lax
from jax.experimental import pallas as pl
from jax.experimental.pallas import tpu as pltpu
from jax.experimental.pallas import tpu_sc as plsc

ROWS = 8192
COLS = 2048
REP = 4
NC = 2
NS = 16
NW = NC * NS
ROWS_PER_W = ROWS // NW
CH = 32
NCHUNK = ROWS_PER_W // CH


def _repeat_kernel(x_hbm, out_hbm, buf, idx0, idx1, idx2, idx3, sem):
    wid = lax.axis_index("s") * NC + lax.axis_index("c")
    base0 = wid * ROWS_PER_W
    idx_refs = (idx0, idx1, idx2, idx3)

    def chunk_body(g, carry):
        base = base0 + g * CH
        pltpu.sync_copy(x_hbm.at[pl.ds(base, CH)], buf)
        for t in range(CH // 16):
            rows = base + t * 16 + lax.iota(jnp.int32, 16)
            for j in range(REP):
                idx_refs[j][pl.ds(t * 16, 16)] = rows * REP + j
        copies = [
            pltpu.async_copy(buf, out_hbm.at[idx_refs[j]], sem)
            for j in range(REP)
        ]
        for c in copies:
            c.wait()
        return carry

    lax.fori_loop(0, NCHUNK, chunk_body, 0)


@jax.jit
def _repeat_interleave(x):
    mesh = plsc.VectorSubcoreMesh(core_axis_name="c", subcore_axis_name="s")
    k = functools.partial(
        pl.kernel,
        out_type=jax.ShapeDtypeStruct((ROWS * REP, COLS), jnp.float32),
        mesh=mesh,
        scratch_types=[
            pltpu.VMEM((CH, COLS), jnp.float32),
            pltpu.VMEM((CH,), jnp.int32),
            pltpu.VMEM((CH,), jnp.int32),
            pltpu.VMEM((CH,), jnp.int32),
            pltpu.VMEM((CH,), jnp.int32),
            pltpu.SemaphoreType.DMA,
        ],
    )(_repeat_kernel)
    return k(x)


def kernel(x):
    return _repeat_interleave(x)
